# baseline (speedup 1.0000x reference)
import jax
import jax.numpy as jnp
from jax.experimental import pallas as pl
from jax.experimental.pallas import tpu as pltpu

_N = 10000
_D = 128
_BM = 400


def _support_body(feature_ref, w_ref, out_ref):
    out_ref[:] = jnp.dot(feature_ref[:], w_ref[:],
                         preferred_element_type=jnp.float32)


def _spmm_body(modal_ref, adj_ref, support_ref, b_ref, feat_ref, out_ref):
    acc = jnp.dot(adj_ref[:], support_ref[:],
                  preferred_element_type=jnp.float32)
    heter = acc + b_ref[:]
    out_ref[:] = jnp.where(modal_ref[0] > 1, heter, feat_ref[:])


def kernel(feature, num_modal, adj_weight, W, b):
    feature = feature.astype(jnp.float32)
    modal = jnp.asarray(num_modal, jnp.int32).reshape(1)
    b2 = b.reshape(1, _D)

    support = pl.pallas_call(
        _support_body,
        out_shape=jax.ShapeDtypeStruct((_N, _D), jnp.float32),
        in_specs=[
            pl.BlockSpec((_N, _D), lambda: (0, 0)),
            pl.BlockSpec((_D, _D), lambda: (0, 0)),
        ],
        out_specs=pl.BlockSpec((_N, _D), lambda: (0, 0)),
    )(feature, W)

    grid_spec = pltpu.PrefetchScalarGridSpec(
        num_scalar_prefetch=1,
        grid=(_N // _BM,),
        in_specs=[
            pl.BlockSpec((_BM, _N), lambda i, modal_ref: (i, 0)),
            pl.BlockSpec((_N, _D), lambda i, modal_ref: (0, 0)),
            pl.BlockSpec((1, _D), lambda i, modal_ref: (0, 0)),
            pl.BlockSpec((_BM, _D), lambda i, modal_ref: (i, 0)),
        ],
        out_specs=pl.BlockSpec((_BM, _D), lambda i, modal_ref: (i, 0)),
    )

    out = pl.pallas_call(
        _spmm_body,
        grid_spec=grid_spec,
        out_shape=jax.ShapeDtypeStruct((_N, _D), jnp.float32),
        compiler_params=pltpu.CompilerParams(
            dimension_semantics=("parallel",),
        ),
    )(modal, adj_weight, support, b2, feature)
    return out


# speedup vs baseline: 1.0571x; 1.0571x over previous
import jax
import jax.numpy as jnp
from jax.experimental import pallas as pl
from jax.experimental.pallas import tpu as pltpu

_N = 10000
_D = 128
_BM = 400


def _body(modal_ref, adj_ref, feature_ref, w_ref, b_ref, out_ref, support_ref):
    i = pl.program_id(0)

    @pl.when(i == 0)
    def _compute_support():
        support_ref[:] = jnp.dot(feature_ref[:], w_ref[:],
                                 preferred_element_type=jnp.float32)

    acc = jnp.dot(adj_ref[:], support_ref[:],
                  preferred_element_type=jnp.float32)
    heter = acc + b_ref[:]
    feat_blk = feature_ref[pl.ds(i * _BM, _BM), :]
    out_ref[:] = jnp.where(modal_ref[0] > 1, heter, feat_blk)


def kernel(feature, num_modal, adj_weight, W, b):
    feature = feature.astype(jnp.float32)
    modal = jnp.asarray(num_modal, jnp.int32).reshape(1)
    b2 = b.reshape(1, _D)

    grid_spec = pltpu.PrefetchScalarGridSpec(
        num_scalar_prefetch=1,
        grid=(_N // _BM,),
        in_specs=[
            pl.BlockSpec((_BM, _N), lambda i, modal_ref: (i, 0)),
            pl.BlockSpec((_N, _D), lambda i, modal_ref: (0, 0)),
            pl.BlockSpec((_D, _D), lambda i, modal_ref: (0, 0)),
            pl.BlockSpec((1, _D), lambda i, modal_ref: (0, 0)),
        ],
        out_specs=pl.BlockSpec((_BM, _D), lambda i, modal_ref: (i, 0)),
        scratch_shapes=[pltpu.VMEM((_N, _D), jnp.float32)],
    )

    out = pl.pallas_call(
        _body,
        grid_spec=grid_spec,
        out_shape=jax.ShapeDtypeStruct((_N, _D), jnp.float32),
        compiler_params=pltpu.CompilerParams(
            dimension_semantics=("arbitrary",),
        ),
    )(modal, adj_weight, feature, W, b2)
    return out
